# baseline (speedup 1.0000x reference)
import functools
import jax
import jax.numpy as jnp
from jax.experimental import pallas as pl
from jax.experimental.pallas import tpu as pltpu

_H = 1024
_FF = 4096
_E = 8
_T = 2048
_BT = 256
_BF = 1024
_FFCH = _FF // _BF
_NTB = _T // _BT


def _dense_moe_body(ei_ref, ew_ref, x_ref, w1_ref, w2_ref, o_ref, acc_ref):
    e = pl.program_id(0)
    kff = pl.program_id(1)
    i = pl.program_id(2)
    rows = pl.ds(i * _BT, _BT)

    @pl.when((e == 0) & (kff == 0))
    def _init():
        acc_ref[rows, :] = jnp.zeros((_BT, _H), jnp.float32)

    h1 = jnp.dot(x_ref[...], w1_ref[0], preferred_element_type=jnp.float32)
    h1 = jax.nn.gelu(h1)
    part = jnp.dot(h1.astype(jnp.bfloat16), w2_ref[0],
                   preferred_element_type=jnp.float32)
    ei = ei_ref[...]
    ew = ew_ref[...]
    gate = jnp.sum(ew * (ei == e).astype(jnp.float32), axis=1)
    acc_ref[rows, :] += gate[:, None] * part

    @pl.when((e == _E - 1) & (kff == _FFCH - 1))
    def _flush():
        o_ref[...] = acc_ref[rows, :]


def kernel(x, scores, expert_weights, expert_indices, w1, w2):
    del scores
    in_shape = x.shape
    tokens = x.reshape(_T, _H).astype(jnp.bfloat16)
    ei = expert_indices.reshape(_T, 2).astype(jnp.int32)
    ew = expert_weights.reshape(_T, 2).astype(jnp.float32)
    w1b = w1.astype(jnp.bfloat16)
    w2b = w2.astype(jnp.bfloat16)

    out = pl.pallas_call(
        _dense_moe_body,
        grid=(_E, _FFCH, _NTB),
        in_specs=[
            pl.BlockSpec((_BT, 2), lambda e, k, i: (i, 0)),
            pl.BlockSpec((_BT, 2), lambda e, k, i: (i, 0)),
            pl.BlockSpec((_BT, _H), lambda e, k, i: (i, 0)),
            pl.BlockSpec((1, _H, _BF), lambda e, k, i: (e, 0, k)),
            pl.BlockSpec((1, _BF, _H), lambda e, k, i: (e, k, 0)),
        ],
        out_specs=pl.BlockSpec((_BT, _H), lambda e, k, i: (i, 0)),
        out_shape=jax.ShapeDtypeStruct((_T, _H), jnp.float32),
        scratch_shapes=[pltpu.VMEM((_T, _H), jnp.float32)],
        compiler_params=pltpu.CompilerParams(
            dimension_semantics=("arbitrary", "arbitrary", "arbitrary"),
        ),
    )(ei, ew, tokens, w1b, w2b)
    return out.reshape(in_shape)
